# baseline (speedup 1.0000x reference)
import functools

import jax
import jax.numpy as jnp
from jax import lax
from jax.experimental import pallas as pl
from jax.experimental.pallas import tpu as pltpu
from jax.experimental.pallas import tpu_sc as plsc

_SPARSITY = 0.2
_L = 16
_W = 2048


@functools.lru_cache(maxsize=None)
def _make_sc_select(batches, n_source, n_target, k):
    num_cores, num_subcores = 2, 16
    nb_local = batches // num_cores
    tpb = num_subcores // nb_local
    slice_rows = n_source // tpb
    rpc = 8
    nchunk = slice_rows // rpc
    nbins = 1 << 16
    hrows = nbins // _W
    qrows = hrows // tpb
    qbins = nbins // tpb
    unroll = 8

    mesh = plsc.VectorSubcoreMesh(core_axis_name="c", subcore_axis_name="s")

    @functools.partial(
        pl.kernel,
        out_type=(
            jax.ShapeDtypeStruct((batches, 8, 128), jnp.float32),
            jax.ShapeDtypeStruct((batches * tpb * (tpb - 1), qrows, _W),
                                 jnp.int32),
            jax.ShapeDtypeStruct((batches * (tpb + 1), 8, 128), jnp.int32),
        ),
        mesh=mesh,
        scratch_types=[
            pltpu.VMEM((rpc, n_target), jnp.float32),
            pltpu.VMEM((rpc, n_target), jnp.float32),
            pltpu.VMEM((hrows, _W), jnp.int32),
            pltpu.VMEM((qrows, _W), jnp.int32),
            pltpu.VMEM((_W,), jnp.int32),
            pltpu.VMEM((tpb, 8, 128), jnp.int32),
            pltpu.VMEM((8, 128), jnp.int32),
            pltpu.VMEM((8, 128), jnp.float32),
            pltpu.SemaphoreType.DMA,
        ],
        compiler_params=pltpu.CompilerParams(
            needs_layout_passes=False, use_tc_tiling_on_sc=True),
    )
    def sc_select(cost_hbm, thr_out, mh_hbm, ms_hbm, buf0, buf1, hist,
                  tmp, tmp2, small4, small1, resf, dsem):
        c = lax.axis_index("c")
        s = lax.axis_index("s")
        lb = s // tpb
        q = s % tpb
        b = c * nb_local + lb
        iota = lax.iota(jnp.int32, _L)
        ones = jnp.ones((_L,), jnp.int32)
        zeros = jnp.zeros((_L,), jnp.int32)
        bufs = (buf0, buf1)
        hq = hist.at[pl.ds(q * qrows, qrows)]

        def zero_hist():
            @plsc.parallel_loop(0, _W, step=_L, unroll=unroll)
            def _(i):
                for r in range(hrows):
                    hist[r, pl.ds(i, _L)] = zeros

        def stream_pass(process16):
            def start(ci, slot):
                pltpu.async_copy(
                    cost_hbm.at[b, pl.ds(q * slice_rows + ci * rpc, rpc), :],
                    bufs[slot].at[...], dsem)

            start(jnp.int32(0), 0)

            def outer(j, _):
                for t in range(2):
                    ci = j * 2 + t

                    @pl.when(ci + 1 < nchunk)
                    def _():
                        start(ci + 1, (t + 1) % 2)

                    pltpu.make_async_copy(
                        cost_hbm.at[b, pl.ds(0, rpc), :],
                        bufs[t].at[...], dsem).wait()

                    @plsc.parallel_loop(0, n_target, step=_L, unroll=unroll)
                    def _(i):
                        for r in range(rpc):
                            process16(bufs[t][r, pl.ds(i, _L)])
                return 0
            lax.fori_loop(0, nchunk // 2, outer, 0)

        def merge_and_scan(rank):
            plsc.subcore_barrier()
            for cq in range(tpb):
                @pl.when(q != cq)
                def _(cq=cq):
                    slot = jnp.where(q < cq, q, q - 1)
                    pltpu.sync_copy(
                        hist.at[pl.ds(cq * qrows, qrows), :],
                        mh_hbm.at[(b * tpb + cq) * (tpb - 1) + slot])
            plsc.subcore_barrier()

            row = (b * tpb + q) * (tpb - 1)
            for slot in range(tpb - 1):
                pltpu.sync_copy(mh_hbm.at[row + slot], tmp)

                @plsc.parallel_loop(0, _W, step=_L, unroll=unroll)
                def _(i):
                    for r in range(qrows):
                        sl = pl.ds(i, _L)
                        hq[r, sl] = hq[r, sl] + tmp[r, sl]

            rsum = []
            for r in range(qrows):
                def rb(i, t16, r=r):
                    return t16 + hq[r, pl.ds(i, _L)]
                t16 = plsc.parallel_loop(
                    0, _W, step=_L, unroll=unroll, carry=zeros)(rb)
                rsum.append(jnp.sum(t16))
            qtot = jnp.int32(0)
            for r in range(qrows):
                qtot = qtot + rsum[r]
            small1[0, pl.ds(0, _L)] = zeros + qtot
            pltpu.sync_copy(small1, ms_hbm.at[b * (tpb + 1) + q])
            plsc.subcore_barrier()

            pltpu.sync_copy(ms_hbm.at[pl.ds(b * (tpb + 1), tpb)], small4)
            qt = [jnp.max(small4[j, 0, pl.ds(0, _L)]) for j in range(tpb)]
            cums = []
            run = jnp.int32(0)
            for j in range(tpb):
                run = run + qt[j]
                cums.append(run)
            owner = jnp.int32(0)
            for j in range(tpb - 1):
                owner = owner + (cums[j] < rank).astype(jnp.int32)
            cumbef = jnp.int32(0)
            for j in range(tpb - 1):
                cumbef = jnp.where(owner == j + 1, cums[j], cumbef)

            @pl.when(q == owner)
            def _():
                rloc = rank - cumbef
                rrun = jnp.int32(0)
                crow = jnp.int32(0)
                runbef = jnp.int32(0)
                fnd = jnp.int32(0)
                for r in range(qrows):
                    newrun = rrun + rsum[r]
                    take = jnp.logical_and(fnd == 0, newrun >= rloc)
                    crow = jnp.where(take, r, crow)
                    runbef = jnp.where(take, rrun, runbef)
                    fnd = jnp.where(take, jnp.int32(1), fnd)
                    rrun = newrun

                @plsc.parallel_loop(0, _W, step=_L, unroll=unroll)
                def _(i):
                    v = hq[0, pl.ds(i, _L)]
                    for r in range(1, qrows):
                        v = jnp.where(crow == r, hq[r, pl.ds(i, _L)], v)
                    tmp2[pl.ds(i, _L)] = v

                def scan_body(i, carry):
                    found, binv, belowv, runv = carry
                    v = tmp2[pl.ds(i * _L, _L)]
                    cs = plsc.cumsum(v)
                    tot = jnp.max(cs)
                    cross = (runv + cs) >= rloc
                    lane = jnp.max(plsc.all_reduce_ffs(cross))
                    below_here = runv + jnp.sum(jnp.where(iota < lane, v, 0))
                    take = jnp.logical_and(found == 0, lane < _L)
                    binv = jnp.where(take, i * _L + lane, binv)
                    belowv = jnp.where(take, below_here, belowv)
                    found = jnp.where(take, jnp.int32(1), found)
                    return (found, binv, belowv, runv + tot)

                _, binv, belowv, _ = lax.fori_loop(
                    0, _W // _L, scan_body,
                    (jnp.int32(0), jnp.int32(0), jnp.int32(0), runbef))
                gbin = q * qbins + crow * _W + binv
                res = jnp.where(iota == 0, gbin,
                                jnp.where(iota == 1, belowv + cumbef, 0))
                small1[0, pl.ds(0, _L)] = res
                pltpu.sync_copy(small1, ms_hbm.at[b * (tpb + 1) + tpb])

            plsc.subcore_barrier()
            pltpu.sync_copy(ms_hbm.at[b * (tpb + 1) + tpb], small1)
            rvec = small1[0, pl.ds(0, _L)]
            bin_out = jnp.sum(jnp.where(iota == 0, rvec, 0))
            below_out = jnp.sum(jnp.where(iota == 1, rvec, 0))
            return bin_out, below_out

        zero_hist()

        def p1(x):
            bits = lax.bitcast_convert_type(x, jnp.int32)
            hi = lax.shift_right_logical(bits, 16)
            plsc.addupdate_scatter(
                hist, [lax.shift_right_logical(hi, 11),
                       jnp.bitwise_and(hi, jnp.int32(_W - 1))], ones)
        stream_pass(p1)
        t_hi, g1 = merge_and_scan(jnp.int32(k))

        zero_hist()

        def p2(x):
            bits = lax.bitcast_convert_type(x, jnp.int32)
            hi = lax.shift_right_logical(bits, 16)
            lo = jnp.bitwise_and(bits, jnp.int32((1 << 16) - 1))
            plsc.addupdate_scatter(
                hist, [lax.shift_right_logical(lo, 11),
                       jnp.bitwise_and(lo, jnp.int32(_W - 1))], ones,
                mask=(hi == t_hi))
        stream_pass(p2)
        t_lo, _ = merge_and_scan(jnp.int32(k) - g1)

        thr_bits = t_hi * jnp.int32(1 << 16) + t_lo

        @pl.when(q == 0)
        def _():
            resf[0, pl.ds(0, _L)] = lax.bitcast_convert_type(
                zeros + thr_bits, jnp.float32)
            pltpu.sync_copy(resf, thr_out.at[b])

    return sc_select


def _mask_kernel(x_ref, t_ref, s_ref, g_ref, o_ref, so_ref, go_ref):
    x = x_ref[...]
    o_ref[...] = jnp.where(x <= t_ref[0, 0], x, 0.0)
    so_ref[...] = s_ref[...]
    go_ref[...] = g_ref[...]


def kernel(source, target, cost_matrix):
    b, n_source, n_target = cost_matrix.shape
    n = n_source * n_target
    k = int(n * _SPARSITY)
    d = source.shape[-1]

    thr_rows, _, _ = _make_sc_select(b, n_source, n_target, k)(cost_matrix)
    thr = thr_rows[:, 0, 0].reshape(b, 1, 1)

    sparse, source_out, target_out = pl.pallas_call(
        _mask_kernel,
        grid=(b,),
        in_specs=[
            pl.BlockSpec((None, n_source, n_target), lambda i: (i, 0, 0)),
            pl.BlockSpec((None, 1, 1), lambda i: (i, 0, 0)),
            pl.BlockSpec((None, n_source, d), lambda i: (i, 0, 0)),
            pl.BlockSpec((None, n_target, d), lambda i: (i, 0, 0)),
        ],
        out_specs=[
            pl.BlockSpec((None, n_source, n_target), lambda i: (i, 0, 0)),
            pl.BlockSpec((None, n_source, d), lambda i: (i, 0, 0)),
            pl.BlockSpec((None, n_target, d), lambda i: (i, 0, 0)),
        ],
        out_shape=[
            jax.ShapeDtypeStruct(cost_matrix.shape, cost_matrix.dtype),
            jax.ShapeDtypeStruct(source.shape, source.dtype),
            jax.ShapeDtypeStruct(target.shape, target.dtype),
        ],
    )(cost_matrix, thr, source, target)
    return (source_out, target_out, sparse)


# speedup vs baseline: 1.0419x; 1.0419x over previous
import functools

import jax
import jax.numpy as jnp
from jax import lax
from jax.experimental import pallas as pl
from jax.experimental.pallas import tpu as pltpu
from jax.experimental.pallas import tpu_sc as plsc

_SPARSITY = 0.2
_L = 16
_W = 2048


@functools.lru_cache(maxsize=None)
def _make_sc_select(batches, n_source, n_target, k):
    num_cores, num_subcores = 2, 16
    nb_local = batches // num_cores
    tpb = num_subcores // nb_local
    slice_rows = n_source // tpb
    rpc = 8
    nchunk = slice_rows // rpc
    nbins = 1 << 16
    hrows = nbins // _W
    qrows = hrows // tpb
    qbins = nbins // tpb
    unroll = 8

    mesh = plsc.VectorSubcoreMesh(core_axis_name="c", subcore_axis_name="s")

    @functools.partial(
        pl.kernel,
        out_type=(
            jax.ShapeDtypeStruct((batches, 8, 128), jnp.float32),
            jax.ShapeDtypeStruct((batches * tpb * (tpb - 1), qrows, _W),
                                 jnp.int32),
            jax.ShapeDtypeStruct((batches * (tpb + 1), 8, 128), jnp.int32),
        ),
        mesh=mesh,
        scratch_types=[
            pltpu.VMEM((rpc, n_target), jnp.float32),
            pltpu.VMEM((rpc, n_target), jnp.float32),
            pltpu.VMEM((hrows, _W), jnp.int32),
            pltpu.VMEM((qrows, _W), jnp.int32),
            pltpu.VMEM((_W,), jnp.int32),
            pltpu.VMEM((tpb, 8, 128), jnp.int32),
            pltpu.VMEM((8, 128), jnp.int32),
            pltpu.VMEM((8, 128), jnp.float32),
            pltpu.SemaphoreType.DMA,
        ],
        compiler_params=pltpu.CompilerParams(
            needs_layout_passes=False, use_tc_tiling_on_sc=True),
    )
    def sc_select(cost_hbm, thr_out, mh_hbm, ms_hbm, buf0, buf1, hist,
                  tmp, tmp2, small4, small1, resf, dsem):
        c = lax.axis_index("c")
        s = lax.axis_index("s")
        lb = s // tpb
        q = s % tpb
        b = c * nb_local + lb
        iota = lax.iota(jnp.int32, _L)
        ones = jnp.ones((_L,), jnp.int32)
        zeros = jnp.zeros((_L,), jnp.int32)
        bufs = (buf0, buf1)
        hq = hist.at[pl.ds(q * qrows, qrows)]

        def zero_hist():
            @plsc.parallel_loop(0, _W, step=_L, unroll=unroll)
            def _(i):
                for r in range(hrows):
                    hist[r, pl.ds(i, _L)] = zeros

        def stream_pass(process16):
            def start(ci, slot):
                pltpu.async_copy(
                    cost_hbm.at[b, pl.ds(q * slice_rows + ci * rpc, rpc), :],
                    bufs[slot].at[...], dsem)

            start(jnp.int32(0), 0)

            def outer(j, _):
                for t in range(2):
                    ci = j * 2 + t

                    @pl.when(ci + 1 < nchunk)
                    def _():
                        start(ci + 1, (t + 1) % 2)

                    pltpu.make_async_copy(
                        cost_hbm.at[b, pl.ds(0, rpc), :],
                        bufs[t].at[...], dsem).wait()

                    @plsc.parallel_loop(0, n_target, step=_L, unroll=2)
                    def _(i):
                        for r in range(rpc):
                            process16(bufs[t][r, pl.ds(i, _L)])
                return 0
            lax.fori_loop(0, nchunk // 2, outer, 0)

        def merge_and_scan(rank):
            plsc.subcore_barrier()
            for cq in range(tpb):
                @pl.when(q != cq)
                def _(cq=cq):
                    slot = jnp.where(q < cq, q, q - 1)
                    pltpu.sync_copy(
                        hist.at[pl.ds(cq * qrows, qrows), :],
                        mh_hbm.at[(b * tpb + cq) * (tpb - 1) + slot])
            plsc.subcore_barrier()

            row = (b * tpb + q) * (tpb - 1)
            for slot in range(tpb - 1):
                pltpu.sync_copy(mh_hbm.at[row + slot], tmp)

                @plsc.parallel_loop(0, _W, step=_L, unroll=unroll)
                def _(i):
                    for r in range(qrows):
                        sl = pl.ds(i, _L)
                        hq[r, sl] = hq[r, sl] + tmp[r, sl]

            rsum = []
            for r in range(qrows):
                def rb(i, t16, r=r):
                    return t16 + hq[r, pl.ds(i, _L)]
                t16 = plsc.parallel_loop(
                    0, _W, step=_L, unroll=unroll, carry=zeros)(rb)
                rsum.append(jnp.sum(t16))
            qtot = jnp.int32(0)
            for r in range(qrows):
                qtot = qtot + rsum[r]
            small1[0, pl.ds(0, _L)] = zeros + qtot
            pltpu.sync_copy(small1, ms_hbm.at[b * (tpb + 1) + q])
            plsc.subcore_barrier()

            pltpu.sync_copy(ms_hbm.at[pl.ds(b * (tpb + 1), tpb)], small4)
            qt = [jnp.max(small4[j, 0, pl.ds(0, _L)]) for j in range(tpb)]
            cums = []
            run = jnp.int32(0)
            for j in range(tpb):
                run = run + qt[j]
                cums.append(run)
            owner = jnp.int32(0)
            for j in range(tpb - 1):
                owner = owner + (cums[j] < rank).astype(jnp.int32)
            cumbef = jnp.int32(0)
            for j in range(tpb - 1):
                cumbef = jnp.where(owner == j + 1, cums[j], cumbef)

            @pl.when(q == owner)
            def _():
                rloc = rank - cumbef
                rrun = jnp.int32(0)
                crow = jnp.int32(0)
                runbef = jnp.int32(0)
                fnd = jnp.int32(0)
                for r in range(qrows):
                    newrun = rrun + rsum[r]
                    take = jnp.logical_and(fnd == 0, newrun >= rloc)
                    crow = jnp.where(take, r, crow)
                    runbef = jnp.where(take, rrun, runbef)
                    fnd = jnp.where(take, jnp.int32(1), fnd)
                    rrun = newrun

                @plsc.parallel_loop(0, _W, step=_L, unroll=unroll)
                def _(i):
                    v = hq[0, pl.ds(i, _L)]
                    for r in range(1, qrows):
                        v = jnp.where(crow == r, hq[r, pl.ds(i, _L)], v)
                    tmp2[pl.ds(i, _L)] = v

                def scan_body(i, carry):
                    found, binv, belowv, runv = carry
                    v = tmp2[pl.ds(i * _L, _L)]
                    cs = plsc.cumsum(v)
                    tot = jnp.max(cs)
                    cross = (runv + cs) >= rloc
                    lane = jnp.max(plsc.all_reduce_ffs(cross))
                    below_here = runv + jnp.sum(jnp.where(iota < lane, v, 0))
                    take = jnp.logical_and(found == 0, lane < _L)
                    binv = jnp.where(take, i * _L + lane, binv)
                    belowv = jnp.where(take, below_here, belowv)
                    found = jnp.where(take, jnp.int32(1), found)
                    return (found, binv, belowv, runv + tot)

                _, binv, belowv, _ = lax.fori_loop(
                    0, _W // _L, scan_body,
                    (jnp.int32(0), jnp.int32(0), jnp.int32(0), runbef))
                gbin = q * qbins + crow * _W + binv
                res = jnp.where(iota == 0, gbin,
                                jnp.where(iota == 1, belowv + cumbef, 0))
                small1[0, pl.ds(0, _L)] = res
                pltpu.sync_copy(small1, ms_hbm.at[b * (tpb + 1) + tpb])

            plsc.subcore_barrier()
            pltpu.sync_copy(ms_hbm.at[b * (tpb + 1) + tpb], small1)
            rvec = small1[0, pl.ds(0, _L)]
            bin_out = jnp.sum(jnp.where(iota == 0, rvec, 0))
            below_out = jnp.sum(jnp.where(iota == 1, rvec, 0))
            return bin_out, below_out

        zero_hist()

        def p1(x):
            bits = lax.bitcast_convert_type(x, jnp.int32)
            hi = lax.shift_right_logical(bits, 16)
            plsc.addupdate_scatter(
                hist, [lax.shift_right_logical(hi, 11),
                       jnp.bitwise_and(hi, jnp.int32(_W - 1))], ones)
        stream_pass(p1)
        t_hi, g1 = merge_and_scan(jnp.int32(k))

        zero_hist()

        def p2(x):
            bits = lax.bitcast_convert_type(x, jnp.int32)
            hi = lax.shift_right_logical(bits, 16)
            lo = jnp.bitwise_and(bits, jnp.int32((1 << 16) - 1))
            plsc.addupdate_scatter(
                hist, [lax.shift_right_logical(lo, 11),
                       jnp.bitwise_and(lo, jnp.int32(_W - 1))], ones,
                mask=(hi == t_hi))
        stream_pass(p2)
        t_lo, _ = merge_and_scan(jnp.int32(k) - g1)

        thr_bits = t_hi * jnp.int32(1 << 16) + t_lo

        @pl.when(q == 0)
        def _():
            resf[0, pl.ds(0, _L)] = lax.bitcast_convert_type(
                zeros + thr_bits, jnp.float32)
            pltpu.sync_copy(resf, thr_out.at[b])

    return sc_select


def _mask_kernel(x_ref, t_ref, s_ref, g_ref, o_ref, so_ref, go_ref):
    x = x_ref[...]
    o_ref[...] = jnp.where(x <= t_ref[0, 0], x, 0.0)
    so_ref[...] = s_ref[...]
    go_ref[...] = g_ref[...]


def kernel(source, target, cost_matrix):
    b, n_source, n_target = cost_matrix.shape
    n = n_source * n_target
    k = int(n * _SPARSITY)
    d = source.shape[-1]

    thr_rows, _, _ = _make_sc_select(b, n_source, n_target, k)(cost_matrix)
    thr = thr_rows[:, 0, 0].reshape(b, 1, 1)

    sparse, source_out, target_out = pl.pallas_call(
        _mask_kernel,
        grid=(b,),
        in_specs=[
            pl.BlockSpec((None, n_source, n_target), lambda i: (i, 0, 0)),
            pl.BlockSpec((None, 1, 1), lambda i: (i, 0, 0)),
            pl.BlockSpec((None, n_source, d), lambda i: (i, 0, 0)),
            pl.BlockSpec((None, n_target, d), lambda i: (i, 0, 0)),
        ],
        out_specs=[
            pl.BlockSpec((None, n_source, n_target), lambda i: (i, 0, 0)),
            pl.BlockSpec((None, n_source, d), lambda i: (i, 0, 0)),
            pl.BlockSpec((None, n_target, d), lambda i: (i, 0, 0)),
        ],
        out_shape=[
            jax.ShapeDtypeStruct(cost_matrix.shape, cost_matrix.dtype),
            jax.ShapeDtypeStruct(source.shape, source.dtype),
            jax.ShapeDtypeStruct(target.shape, target.dtype),
        ],
    )(cost_matrix, thr, source, target)
    return (source_out, target_out, sparse)


# speedup vs baseline: 1.0441x; 1.0021x over previous
import functools

import jax
import jax.numpy as jnp
from jax import lax
from jax.experimental import pallas as pl
from jax.experimental.pallas import tpu as pltpu
from jax.experimental.pallas import tpu_sc as plsc

_SPARSITY = 0.2
_L = 16
_W = 2048


@functools.lru_cache(maxsize=None)
def _make_sc_select(batches, n_source, n_target, k):
    num_cores, num_subcores = 2, 16
    nb_local = batches // num_cores
    tpb = num_subcores // nb_local
    slice_rows = n_source // tpb
    rpc = 8
    nchunk = slice_rows // rpc
    nbins = 1 << 16
    hrows = nbins // _W
    qrows = hrows // tpb
    qbins = nbins // tpb
    unroll = 8

    mesh = plsc.VectorSubcoreMesh(core_axis_name="c", subcore_axis_name="s")

    @functools.partial(
        pl.kernel,
        out_type=(
            jax.ShapeDtypeStruct((batches, 8, 128), jnp.float32),
            jax.ShapeDtypeStruct((batches * tpb * (tpb - 1), qrows, _W),
                                 jnp.int32),
            jax.ShapeDtypeStruct((batches * (tpb + 1), 8, 128), jnp.int32),
        ),
        mesh=mesh,
        scratch_types=[
            pltpu.VMEM((rpc, n_target), jnp.float32),
            pltpu.VMEM((rpc, n_target), jnp.float32),
            pltpu.VMEM((hrows, _W), jnp.int32),
            pltpu.VMEM((qrows, _W), jnp.int32),
            pltpu.VMEM((_W,), jnp.int32),
            pltpu.VMEM((tpb, 8, 128), jnp.int32),
            pltpu.VMEM((8, 128), jnp.int32),
            pltpu.VMEM((8, 128), jnp.float32),
            pltpu.SemaphoreType.DMA,
        ],
        compiler_params=pltpu.CompilerParams(
            needs_layout_passes=False, use_tc_tiling_on_sc=True),
    )
    def sc_select(cost_hbm, thr_out, mh_hbm, ms_hbm, buf0, buf1, hist,
                  tmp, tmp2, small4, small1, resf, dsem):
        c = lax.axis_index("c")
        s = lax.axis_index("s")
        lb = s // tpb
        q = s % tpb
        b = c * nb_local + lb
        iota = lax.iota(jnp.int32, _L)
        ones = jnp.ones((_L,), jnp.int32)
        zeros = jnp.zeros((_L,), jnp.int32)
        bufs = (buf0, buf1)
        hq = hist.at[pl.ds(q * qrows, qrows)]

        def zero_hist():
            @plsc.parallel_loop(0, _W, step=_L, unroll=unroll)
            def _(i):
                for r in range(hrows):
                    hist[r, pl.ds(i, _L)] = zeros

        def stream_pass(process16):
            def start(ci, slot):
                pltpu.async_copy(
                    cost_hbm.at[b, pl.ds(q * slice_rows + ci * rpc, rpc), :],
                    bufs[slot].at[...], dsem)

            start(jnp.int32(0), 0)

            def outer(j, _):
                for t in range(2):
                    ci = j * 2 + t

                    @pl.when(ci + 1 < nchunk)
                    def _():
                        start(ci + 1, (t + 1) % 2)

                    pltpu.make_async_copy(
                        cost_hbm.at[b, pl.ds(0, rpc), :],
                        bufs[t].at[...], dsem).wait()

                    @plsc.parallel_loop(0, n_target, step=_L, unroll=4)
                    def _(i):
                        for r in range(rpc):
                            process16(bufs[t][r, pl.ds(i, _L)])
                return 0
            lax.fori_loop(0, nchunk // 2, outer, 0)

        def merge_and_scan(rank):
            plsc.subcore_barrier()
            for cq in range(tpb):
                @pl.when(q != cq)
                def _(cq=cq):
                    slot = jnp.where(q < cq, q, q - 1)
                    pltpu.sync_copy(
                        hist.at[pl.ds(cq * qrows, qrows), :],
                        mh_hbm.at[(b * tpb + cq) * (tpb - 1) + slot])
            plsc.subcore_barrier()

            row = (b * tpb + q) * (tpb - 1)
            for slot in range(tpb - 1):
                pltpu.sync_copy(mh_hbm.at[row + slot], tmp)

                @plsc.parallel_loop(0, _W, step=_L, unroll=unroll)
                def _(i):
                    for r in range(qrows):
                        sl = pl.ds(i, _L)
                        hq[r, sl] = hq[r, sl] + tmp[r, sl]

            rsum = []
            for r in range(qrows):
                def rb(i, t16, r=r):
                    return t16 + hq[r, pl.ds(i, _L)]
                t16 = plsc.parallel_loop(
                    0, _W, step=_L, unroll=unroll, carry=zeros)(rb)
                rsum.append(jnp.sum(t16))
            qtot = jnp.int32(0)
            for r in range(qrows):
                qtot = qtot + rsum[r]
            small1[0, pl.ds(0, _L)] = zeros + qtot
            pltpu.sync_copy(small1, ms_hbm.at[b * (tpb + 1) + q])
            plsc.subcore_barrier()

            pltpu.sync_copy(ms_hbm.at[pl.ds(b * (tpb + 1), tpb)], small4)
            qt = [jnp.max(small4[j, 0, pl.ds(0, _L)]) for j in range(tpb)]
            cums = []
            run = jnp.int32(0)
            for j in range(tpb):
                run = run + qt[j]
                cums.append(run)
            owner = jnp.int32(0)
            for j in range(tpb - 1):
                owner = owner + (cums[j] < rank).astype(jnp.int32)
            cumbef = jnp.int32(0)
            for j in range(tpb - 1):
                cumbef = jnp.where(owner == j + 1, cums[j], cumbef)

            @pl.when(q == owner)
            def _():
                rloc = rank - cumbef
                rrun = jnp.int32(0)
                crow = jnp.int32(0)
                runbef = jnp.int32(0)
                fnd = jnp.int32(0)
                for r in range(qrows):
                    newrun = rrun + rsum[r]
                    take = jnp.logical_and(fnd == 0, newrun >= rloc)
                    crow = jnp.where(take, r, crow)
                    runbef = jnp.where(take, rrun, runbef)
                    fnd = jnp.where(take, jnp.int32(1), fnd)
                    rrun = newrun

                @plsc.parallel_loop(0, _W, step=_L, unroll=unroll)
                def _(i):
                    v = hq[0, pl.ds(i, _L)]
                    for r in range(1, qrows):
                        v = jnp.where(crow == r, hq[r, pl.ds(i, _L)], v)
                    tmp2[pl.ds(i, _L)] = v

                def scan_body(i, carry):
                    found, binv, belowv, runv = carry
                    v = tmp2[pl.ds(i * _L, _L)]
                    cs = plsc.cumsum(v)
                    tot = jnp.max(cs)
                    cross = (runv + cs) >= rloc
                    lane = jnp.max(plsc.all_reduce_ffs(cross))
                    below_here = runv + jnp.sum(jnp.where(iota < lane, v, 0))
                    take = jnp.logical_and(found == 0, lane < _L)
                    binv = jnp.where(take, i * _L + lane, binv)
                    belowv = jnp.where(take, below_here, belowv)
                    found = jnp.where(take, jnp.int32(1), found)
                    return (found, binv, belowv, runv + tot)

                _, binv, belowv, _ = lax.fori_loop(
                    0, _W // _L, scan_body,
                    (jnp.int32(0), jnp.int32(0), jnp.int32(0), runbef))
                gbin = q * qbins + crow * _W + binv
                res = jnp.where(iota == 0, gbin,
                                jnp.where(iota == 1, belowv + cumbef, 0))
                small1[0, pl.ds(0, _L)] = res
                pltpu.sync_copy(small1, ms_hbm.at[b * (tpb + 1) + tpb])

            plsc.subcore_barrier()
            pltpu.sync_copy(ms_hbm.at[b * (tpb + 1) + tpb], small1)
            rvec = small1[0, pl.ds(0, _L)]
            bin_out = jnp.sum(jnp.where(iota == 0, rvec, 0))
            below_out = jnp.sum(jnp.where(iota == 1, rvec, 0))
            return bin_out, below_out

        zero_hist()

        def p1(x):
            bits = lax.bitcast_convert_type(x, jnp.int32)
            hi = lax.shift_right_logical(bits, 16)
            plsc.addupdate_scatter(
                hist, [lax.shift_right_logical(hi, 11),
                       jnp.bitwise_and(hi, jnp.int32(_W - 1))], ones)
        stream_pass(p1)
        t_hi, g1 = merge_and_scan(jnp.int32(k))

        zero_hist()

        def p2(x):
            bits = lax.bitcast_convert_type(x, jnp.int32)
            hi = lax.shift_right_logical(bits, 16)
            lo = jnp.bitwise_and(bits, jnp.int32((1 << 16) - 1))
            plsc.addupdate_scatter(
                hist, [lax.shift_right_logical(lo, 11),
                       jnp.bitwise_and(lo, jnp.int32(_W - 1))], ones,
                mask=(hi == t_hi))
        stream_pass(p2)
        t_lo, _ = merge_and_scan(jnp.int32(k) - g1)

        thr_bits = t_hi * jnp.int32(1 << 16) + t_lo

        @pl.when(q == 0)
        def _():
            resf[0, pl.ds(0, _L)] = lax.bitcast_convert_type(
                zeros + thr_bits, jnp.float32)
            pltpu.sync_copy(resf, thr_out.at[b])

    return sc_select


def _mask_kernel(x_ref, t_ref, s_ref, g_ref, o_ref, so_ref, go_ref):
    x = x_ref[...]
    o_ref[...] = jnp.where(x <= t_ref[0, 0], x, 0.0)
    so_ref[...] = s_ref[...]
    go_ref[...] = g_ref[...]


def kernel(source, target, cost_matrix):
    b, n_source, n_target = cost_matrix.shape
    n = n_source * n_target
    k = int(n * _SPARSITY)
    d = source.shape[-1]

    thr_rows, _, _ = _make_sc_select(b, n_source, n_target, k)(cost_matrix)
    thr = thr_rows[:, 0, 0].reshape(b, 1, 1)

    sparse, source_out, target_out = pl.pallas_call(
        _mask_kernel,
        grid=(b,),
        in_specs=[
            pl.BlockSpec((None, n_source, n_target), lambda i: (i, 0, 0)),
            pl.BlockSpec((None, 1, 1), lambda i: (i, 0, 0)),
            pl.BlockSpec((None, n_source, d), lambda i: (i, 0, 0)),
            pl.BlockSpec((None, n_target, d), lambda i: (i, 0, 0)),
        ],
        out_specs=[
            pl.BlockSpec((None, n_source, n_target), lambda i: (i, 0, 0)),
            pl.BlockSpec((None, n_source, d), lambda i: (i, 0, 0)),
            pl.BlockSpec((None, n_target, d), lambda i: (i, 0, 0)),
        ],
        out_shape=[
            jax.ShapeDtypeStruct(cost_matrix.shape, cost_matrix.dtype),
            jax.ShapeDtypeStruct(source.shape, source.dtype),
            jax.ShapeDtypeStruct(target.shape, target.dtype),
        ],
    )(cost_matrix, thr, source, target)
    return (source_out, target_out, sparse)
